# baseline (speedup 1.0000x reference)
import jax
import jax.numpy as jnp
from jax.experimental import pallas as pl
from jax.experimental.pallas import tpu as pltpu

H = 56
D2 = 112
NSTEP = 3
S2S_STEPS = 2
NB = 64
N = 2048
E = 16384


def _mpnn(x, ei, ea, p):
    h = jax.nn.relu(x @ p['lin0_w'] + p['lin0_b'])
    eh = jax.nn.relu(ea @ p['e1_w'] + p['e1_b'])
    Wk = jnp.concatenate([p['e2_w'], p['e2_b'][None, :]], axis=0)
    Wcat = Wk.reshape(11, H, H).transpose(1, 0, 2).reshape(H, 11 * H)
    src = ei[0]
    dst = ei[1]
    ehaug = jnp.concatenate([eh, jnp.ones((E, 1), eh.dtype)], axis=1)
    for _ in range(NSTEP):
        q = (h @ Wcat).reshape(N, 11, H)
        qsrc = q[src]
        msg = jnp.einsum('ek,eko->eo', ehaug, qsrc)
        agg = jax.ops.segment_sum(msg, dst, num_segments=N)
        m = jax.nn.relu(h @ p['root_w'] + agg)
        gi = m @ p['gru_wih'].T + p['gru_bih']
        gh = h @ p['gru_whh'].T + p['gru_bhh']
        ir, iz, inn = jnp.split(gi, 3, axis=1)
        hr, hz, hn = jnp.split(gh, 3, axis=1)
        r = jax.nn.sigmoid(ir + hr)
        z = jax.nn.sigmoid(iz + hz)
        n = jnp.tanh(inn + r * hn)
        h = (1.0 - z) * n + z * h
    return h


def _normalize(x):
    return x / jnp.maximum(jnp.linalg.norm(x, axis=1, keepdims=True), 1e-12)


def _seg_softmax(e, seg, nb):
    m = jax.ops.segment_max(e, seg, num_segments=nb)
    m = jnp.where(jnp.isfinite(m), m, 0.0)
    ex = jnp.exp(e - m[seg])
    s = jax.ops.segment_sum(ex, seg, num_segments=nb)
    return ex / (s[seg] + 1e-16)


def _set2set(x, seg, nb, p):
    d = x.shape[1]
    q_star = jnp.zeros((nb, 2 * d), dtype=x.dtype)
    h = jnp.zeros((nb, d), dtype=x.dtype)
    cst = jnp.zeros((nb, d), dtype=x.dtype)
    for _ in range(S2S_STEPS):
        gates = q_star @ p['wih'].T + p['bih'] + h @ p['whh'].T + p['bhh']
        i, f, g, o = jnp.split(gates, 4, axis=1)
        i = jax.nn.sigmoid(i)
        f = jax.nn.sigmoid(f)
        g = jnp.tanh(g)
        o = jax.nn.sigmoid(o)
        cst = f * cst + i * g
        h = o * jnp.tanh(cst)
        e = jnp.sum(x * h[seg], axis=1)
        a = _seg_softmax(e, seg, nb)
        r = jax.ops.segment_sum(a[:, None] * x, seg, num_segments=nb)
        q_star = jnp.concatenate([h, r], axis=1)
    return q_star


def _seg_mean_std(x, seg, nb):
    ones = jnp.ones((x.shape[0], 1), dtype=x.dtype)
    cnt = jax.ops.segment_sum(ones, seg, num_segments=nb)
    s = jax.ops.segment_sum(x, seg, num_segments=nb)
    mean = s / jnp.maximum(cnt, 1.0)
    s2 = jax.ops.segment_sum(x * x, seg, num_segments=nb)
    var = (s2 - cnt * mean * mean) / jnp.maximum(cnt - 1.0, 1.0)
    std = jnp.sqrt(jnp.maximum(var, 0.0))
    return mean, std


def _contrastive(a, b, tau):
    na = jnp.linalg.norm(a, axis=1)
    nb_ = jnp.linalg.norm(b, axis=1)
    sim = (a @ b.T) / (na[:, None] * nb_[None, :])
    sim = jnp.exp(sim / tau)
    pos = jnp.diag(sim)
    loss = pos / (sim.sum(axis=1) - pos)
    return -jnp.log(loss).mean()


def _pred_mlp_kernel(x_ref, w1_ref, b1_ref, w2_ref, b2_ref, w3_ref, b3_ref,
                     out_ref):
    h1 = jax.nn.relu(x_ref[...] @ w1_ref[...] + b1_ref[...])
    h2 = jax.nn.relu(h1 @ w2_ref[...] + b2_ref[...])
    out_ref[...] = h2 @ w3_ref[...] + b3_ref[...]


def _pred_mlp(final, pr):
    return pl.pallas_call(
        _pred_mlp_kernel,
        out_shape=jax.ShapeDtypeStruct((final.shape[0], 1), final.dtype),
    )(final, pr['p1_w'], pr['p1_b'][None, :], pr['p2_w'], pr['p2_b'][None, :],
      pr['p3_w'], pr['p3_b'][None, :])


def kernel(solute_x, solute_edge_index, solute_edge_attr, solute_batch,
           solvent_x, solvent_edge_index, solvent_edge_attr, solvent_batch,
           params):
    sx, sei, sea, sb = solute_x, solute_edge_index, solute_edge_attr, solute_batch
    vx, vei, vea, vb = solvent_x, solvent_edge_index, solvent_edge_attr, solvent_batch
    rng = jax.random.key(7)
    hs = _mpnn(sx, sei, sea, params['solute'])
    hv = _mpnn(vx, vei, vea, params['solvent'])
    fs = _normalize(hs)
    fv = _normalize(hv)
    len_map = (sb[:, None] == vb[None, :]).astype(fs.dtype)
    imap = (fs @ fv.T) * len_map
    v_prime = imap.T @ fs
    s_prime = imap @ fv
    fs = jnp.concatenate([fs, s_prime], axis=1)
    fv = jnp.concatenate([fv, v_prime], axis=1)
    c = params['compressor']
    a = fs @ c['c1_w'] + c['c1_b']
    mu = a.mean(axis=0)
    var = a.var(axis=0)
    a = (a - mu) / jnp.sqrt(var + 1e-5) * c['bn_g'] + c['bn_b']
    a = jax.nn.relu(a)
    p_logit = a @ c['c2_w'] + c['c2_b']
    k1, k2 = jax.random.split(rng)
    bias = 1e-4
    u = jax.random.uniform(k1, p_logit.shape, dtype=p_logit.dtype)
    eps = (2.0 * bias - 1.0) * u + (1.0 - bias)
    gate = jax.nn.sigmoid(jnp.log(eps) - jnp.log(1.0 - eps) + p_logit)
    lam_pos = gate.reshape(-1, 1)
    lam_neg = 1.0 - lam_pos
    static = fs
    mean_g, std_g = _seg_mean_std(static, sb, NB)
    mean_n = mean_g[sb]
    std_n = std_g[sb]
    noisy_mean = lam_pos * fs + lam_neg * mean_n
    noisy_std = lam_neg * std_n
    noise = jax.random.uniform(k2, noisy_mean.shape, dtype=noisy_mean.dtype)
    noisy = noisy_mean + noise * noisy_std
    sub_s = _set2set(noisy, sb, NB, params['s2s_solute'])
    eps2 = 1e-07
    kl1 = jax.ops.segment_sum(((noisy_std ** 2) / ((std_n + eps2) ** 2)).mean(axis=1), sb, num_segments=NB).reshape(-1, 1)
    kl2 = jax.ops.segment_sum(((noisy_mean - mean_n) / (std_n + eps2)) ** 2, sb, num_segments=NB)
    KL_Loss = (0.5 * kl1 + kl2).mean()
    sub_v = _set2set(fv, vb, NB, params['s2s_solvent'])
    cont = _contrastive(sub_s, sub_v, 1.0)
    final = jnp.concatenate([sub_s, sub_v], axis=1)
    preds = _pred_mlp(final, params['pred'])
    return preds, KL_Loss, cont


# speedup vs baseline: 1.1411x; 1.1411x over previous
import functools

import jax
import jax.numpy as jnp
from jax.experimental import pallas as pl
from jax.experimental.pallas import tpu as pltpu

H = 56
HP = 64
D2 = 112
NSTEP = 3
S2S_STEPS = 2
NB = 64
N = 2048
E = 16384
NK = 11
BE = 2048


def _msg_body(ehaug_ref, hsrc_ref, w_ref, out_ref, g_ref):
    hs = hsrc_ref[...]
    for k in range(NK):
        g_ref[:, k * HP:(k + 1) * HP] = ehaug_ref[:, k:k + 1] * hs
    out_ref[...] = jnp.dot(g_ref[...], w_ref[...],
                           preferred_element_type=jnp.float32,
                           precision=jax.lax.Precision.HIGHEST)


def _msg_kernel(ehaug, hsrc, wstack):
    return pl.pallas_call(
        _msg_body,
        grid=(E // BE,),
        in_specs=[
            pl.BlockSpec((BE, 16), lambda i: (i, 0)),
            pl.BlockSpec((BE, HP), lambda i: (i, 0)),
            pl.BlockSpec((NK * HP, HP), lambda i: (0, 0)),
        ],
        out_specs=pl.BlockSpec((BE, HP), lambda i: (i, 0)),
        out_shape=jax.ShapeDtypeStruct((E, HP), jnp.float32),
        scratch_shapes=[pltpu.VMEM((BE, NK * HP), jnp.float32)],
    )(ehaug, hsrc, wstack)


def _gru_body(h_ref, agg_ref, root_ref, wih_ref, whh_ref, bih_ref, bhh_ref,
              out_ref):
    h = h_ref[...]
    m = jax.nn.relu(jnp.dot(h, root_ref[...],
                            preferred_element_type=jnp.float32) + agg_ref[...])
    gi = jnp.dot(m, wih_ref[...], preferred_element_type=jnp.float32) \
        + bih_ref[...]
    gh = jnp.dot(h, whh_ref[...], preferred_element_type=jnp.float32) \
        + bhh_ref[...]
    ir, iz, inn = gi[:, :HP], gi[:, HP:2 * HP], gi[:, 2 * HP:]
    hr, hz, hn = gh[:, :HP], gh[:, HP:2 * HP], gh[:, 2 * HP:]
    r = jax.nn.sigmoid(ir + hr)
    z = jax.nn.sigmoid(iz + hz)
    n = jnp.tanh(inn + r * hn)
    hnew = (1.0 - z) * n + z * h
    out_ref[...] = hnew


def _gru_kernel(h, agg, root, wih, whh, bih, bhh):
    return pl.pallas_call(
        _gru_body,
        out_shape=jax.ShapeDtypeStruct((N, HP), jnp.float32),
    )(h, agg, root, wih, whh, bih, bhh)


def _prologue_body(x_ref, w_ref, b_ref, out_ref):
    out_ref[...] = jax.nn.relu(
        jnp.dot(x_ref[...], w_ref[...], preferred_element_type=jnp.float32)
        + b_ref[...])


def _lin_relu(x, w, b, m, nout):
    return pl.pallas_call(
        _prologue_body,
        out_shape=jax.ShapeDtypeStruct((m, nout), jnp.float32),
    )(x, w, b)


def _ehaug_body(ea_ref, w_ref, b_ref, out_ref):
    eh = jax.nn.relu(
        jnp.dot(ea_ref[...], w_ref[...], preferred_element_type=jnp.float32)
        + b_ref[...])
    col = jax.lax.broadcasted_iota(jnp.int32, eh.shape, 1)
    eh_r = eh.astype(jnp.bfloat16).astype(jnp.float32)
    out_ref[...] = eh_r + jnp.where(col == 10, 1.0, 0.0)


def _ehaug_kernel(ea_pad, w, b):
    return pl.pallas_call(
        _ehaug_body,
        grid=(E // BE,),
        in_specs=[
            pl.BlockSpec((BE, 16), lambda i: (i, 0)),
            pl.BlockSpec((16, 16), lambda i: (0, 0)),
            pl.BlockSpec((1, 16), lambda i: (0, 0)),
        ],
        out_specs=pl.BlockSpec((BE, 16), lambda i: (i, 0)),
        out_shape=jax.ShapeDtypeStruct((E, 16), jnp.float32),
    )(ea_pad, w, b)


def _pad2(w, r, c):
    return jnp.zeros((r, c), w.dtype).at[:w.shape[0], :w.shape[1]].set(w)


def _pack_mpnn(p):
    q = {}
    q['lin0_w'] = _pad2(p['lin0_w'], HP, HP)
    q['lin0_b'] = _pad2(p['lin0_b'][None, :], 1, HP)
    q['e1_w'] = _pad2(p['e1_w'], 16, 16)
    q['e1_b'] = _pad2(p['e1_b'][None, :], 1, 16)
    e2w_r = p['e2_w'].astype(jnp.bfloat16).astype(jnp.float32)
    wk = jnp.concatenate([e2w_r, p['e2_b'][None, :]], axis=0)
    wk = wk.reshape(NK, H, H)
    q['wstack'] = jnp.zeros((NK, HP, HP), jnp.float32) \
        .at[:, :H, :H].set(wk).reshape(NK * HP, HP)
    q['root_w'] = _pad2(p['root_w'], HP, HP)
    wih = p['gru_wih'].T.reshape(H, 3, H)
    q['wih'] = jnp.zeros((HP, 3, HP), jnp.float32) \
        .at[:H, :, :H].set(wih).reshape(HP, 3 * HP)
    whh = p['gru_whh'].T.reshape(H, 3, H)
    q['whh'] = jnp.zeros((HP, 3, HP), jnp.float32) \
        .at[:H, :, :H].set(whh).reshape(HP, 3 * HP)
    bih = p['gru_bih'].reshape(3, H)
    q['bih'] = jnp.zeros((1, 3, HP), jnp.float32) \
        .at[0, :, :H].set(bih).reshape(1, 3 * HP)
    bhh = p['gru_bhh'].reshape(3, H)
    q['bhh'] = jnp.zeros((1, 3, HP), jnp.float32) \
        .at[0, :, :H].set(bhh).reshape(1, 3 * HP)
    return q


def _mpnn(x_pad, ea_pad, src, dst, q):
    h = _lin_relu(x_pad, q['lin0_w'], q['lin0_b'], N, HP)
    ehaug = _ehaug_kernel(ea_pad, q['e1_w'], q['e1_b'])
    for _ in range(NSTEP):
        hsrc = jnp.take(h, src, axis=0)
        msg = _msg_kernel(ehaug, hsrc, q['wstack'])
        agg = jax.ops.segment_sum(msg, dst, num_segments=N)
        h = _gru_kernel(h, agg, q['root_w'], q['wih'], q['whh'],
                        q['bih'], q['bhh'])
    return h[:, :H]


def _normalize(x):
    return x / jnp.maximum(jnp.linalg.norm(x, axis=1, keepdims=True), 1e-12)


def _seg_softmax(e, seg, nb):
    m = jax.ops.segment_max(e, seg, num_segments=nb)
    m = jnp.where(jnp.isfinite(m), m, 0.0)
    ex = jnp.exp(e - m[seg])
    s = jax.ops.segment_sum(ex, seg, num_segments=nb)
    return ex / (s[seg] + 1e-16)


def _set2set(x, seg, nb, p):
    d = x.shape[1]
    q_star = jnp.zeros((nb, 2 * d), dtype=x.dtype)
    h = jnp.zeros((nb, d), dtype=x.dtype)
    cst = jnp.zeros((nb, d), dtype=x.dtype)
    for _ in range(S2S_STEPS):
        gates = q_star @ p['wih'].T + p['bih'] + h @ p['whh'].T + p['bhh']
        i, f, g, o = jnp.split(gates, 4, axis=1)
        i = jax.nn.sigmoid(i)
        f = jax.nn.sigmoid(f)
        g = jnp.tanh(g)
        o = jax.nn.sigmoid(o)
        cst = f * cst + i * g
        h = o * jnp.tanh(cst)
        e = jnp.sum(x * h[seg], axis=1)
        a = _seg_softmax(e, seg, nb)
        r = jax.ops.segment_sum(a[:, None] * x, seg, num_segments=nb)
        q_star = jnp.concatenate([h, r], axis=1)
    return q_star


def _seg_mean_std(x, seg, nb):
    ones = jnp.ones((x.shape[0], 1), dtype=x.dtype)
    cnt = jax.ops.segment_sum(ones, seg, num_segments=nb)
    s = jax.ops.segment_sum(x, seg, num_segments=nb)
    mean = s / jnp.maximum(cnt, 1.0)
    s2 = jax.ops.segment_sum(x * x, seg, num_segments=nb)
    var = (s2 - cnt * mean * mean) / jnp.maximum(cnt - 1.0, 1.0)
    std = jnp.sqrt(jnp.maximum(var, 0.0))
    return mean, std


def _contrastive(a, b, tau):
    na = jnp.linalg.norm(a, axis=1)
    nb_ = jnp.linalg.norm(b, axis=1)
    sim = (a @ b.T) / (na[:, None] * nb_[None, :])
    sim = jnp.exp(sim / tau)
    pos = jnp.diag(sim)
    loss = pos / (sim.sum(axis=1) - pos)
    return -jnp.log(loss).mean()


def _pred_mlp_body(x_ref, w1_ref, b1_ref, w2_ref, b2_ref, w3_ref, b3_ref,
                   out_ref):
    h1 = jax.nn.relu(x_ref[...] @ w1_ref[...] + b1_ref[...])
    h2 = jax.nn.relu(h1 @ w2_ref[...] + b2_ref[...])
    out_ref[...] = h2 @ w3_ref[...] + b3_ref[...]


def _pred_mlp(final, pr):
    return pl.pallas_call(
        _pred_mlp_body,
        out_shape=jax.ShapeDtypeStruct((final.shape[0], 1), final.dtype),
    )(final, pr['p1_w'], pr['p1_b'][None, :], pr['p2_w'], pr['p2_b'][None, :],
      pr['p3_w'], pr['p3_b'][None, :])


def kernel(solute_x, solute_edge_index, solute_edge_attr, solute_batch,
           solvent_x, solvent_edge_index, solvent_edge_attr, solvent_batch,
           params):
    sb, vb = solute_batch, solvent_batch
    rng = jax.random.key(7)

    def padx(x):
        return jnp.zeros((N, HP), x.dtype).at[:, :H].set(x)

    def padea(ea):
        return jnp.zeros((E, 16), ea.dtype).at[:, :10].set(ea)

    qs = _pack_mpnn(params['solute'])
    qv = _pack_mpnn(params['solvent'])
    hs = _mpnn(padx(solute_x), padea(solute_edge_attr),
               solute_edge_index[0], solute_edge_index[1], qs)
    hv = _mpnn(padx(solvent_x), padea(solvent_edge_attr),
               solvent_edge_index[0], solvent_edge_index[1], qv)
    fs = _normalize(hs)
    fv = _normalize(hv)
    len_map = (sb[:, None] == vb[None, :]).astype(fs.dtype)
    imap = (fs @ fv.T) * len_map
    v_prime = imap.T @ fs
    s_prime = imap @ fv
    fs = jnp.concatenate([fs, s_prime], axis=1)
    fv = jnp.concatenate([fv, v_prime], axis=1)
    c = params['compressor']
    a = fs @ c['c1_w'] + c['c1_b']
    mu = a.mean(axis=0)
    var = a.var(axis=0)
    a = (a - mu) / jnp.sqrt(var + 1e-5) * c['bn_g'] + c['bn_b']
    a = jax.nn.relu(a)
    p_logit = a @ c['c2_w'] + c['c2_b']
    k1, k2 = jax.random.split(rng)
    bias = 1e-4
    u = jax.random.uniform(k1, p_logit.shape, dtype=p_logit.dtype)
    eps = (2.0 * bias - 1.0) * u + (1.0 - bias)
    gate = jax.nn.sigmoid(jnp.log(eps) - jnp.log(1.0 - eps) + p_logit)
    lam_pos = gate.reshape(-1, 1)
    lam_neg = 1.0 - lam_pos
    mean_g, std_g = _seg_mean_std(fs, sb, NB)
    mean_n = mean_g[sb]
    std_n = std_g[sb]
    noisy_mean = lam_pos * fs + lam_neg * mean_n
    noisy_std = lam_neg * std_n
    noise = jax.random.uniform(k2, noisy_mean.shape, dtype=noisy_mean.dtype)
    noisy = noisy_mean + noise * noisy_std
    sub_s = _set2set(noisy, sb, NB, params['s2s_solute'])
    eps2 = 1e-07
    kl1 = jax.ops.segment_sum(((noisy_std ** 2) / ((std_n + eps2) ** 2)).mean(axis=1), sb, num_segments=NB).reshape(-1, 1)
    kl2 = jax.ops.segment_sum(((noisy_mean - mean_n) / (std_n + eps2)) ** 2, sb, num_segments=NB)
    KL_Loss = (0.5 * kl1 + kl2).mean()
    sub_v = _set2set(fv, vb, NB, params['s2s_solvent'])
    cont = _contrastive(sub_s, sub_v, 1.0)
    final = jnp.concatenate([sub_s, sub_v], axis=1)
    preds = _pred_mlp(final, params['pred'])
    return preds, KL_Loss, cont


# speedup vs baseline: 1.4909x; 1.3066x over previous
import functools

import jax
import jax.numpy as jnp
from jax import lax
from jax.experimental import pallas as pl
from jax.experimental.pallas import tpu as pltpu
from jax.experimental.pallas import tpu_sc as plsc

H = 56
HP = 64
D2 = 112
NSTEP = 3
S2S_STEPS = 2
NB = 64
N = 2048
E = 16384
NK = 11
HG = 128
BE = 2048


_NC = 2
_NS = 16
_NW = _NC * _NS
_EPW = E // _NW
_CH = _EPW // 128

_sc_mesh = plsc.VectorSubcoreMesh(core_axis_name="c", subcore_axis_name="s")


@functools.partial(
    pl.kernel,
    mesh=_sc_mesh,
    out_type=jax.ShapeDtypeStruct((E, HG), jnp.float32),
    scratch_types=[
        pltpu.VMEM((_CH, 128), jnp.int32),
        pltpu.VMEM((_EPW, HG), jnp.float32),
        pltpu.SemaphoreType.DMA,
    ],
)
def _sc_gather(h_hbm, idx_hbm, out_hbm, idx_v, rows_v, sem):
    wid = lax.axis_index("s") * _NC + lax.axis_index("c")
    pltpu.sync_copy(idx_hbm.at[pl.ds(wid * _CH, _CH)], idx_v)
    copies = []
    for j in range(_CH):
        copies.append(pltpu.async_copy(
            h_hbm.at[idx_v.at[j]], rows_v.at[pl.ds(j * 128, 128)], sem))
    for cp in copies:
        cp.wait()
    pltpu.sync_copy(rows_v, out_hbm.at[pl.ds(wid * _EPW, _EPW)])


@functools.partial(
    pl.kernel,
    mesh=_sc_mesh,
    out_type=jax.ShapeDtypeStruct((_NC * N, HG), jnp.float32),
    scratch_types=[
        pltpu.VMEM((_CH, 128), jnp.int32),
        pltpu.VMEM((_EPW, HG), jnp.float32),
        pltpu.VMEM_SHARED((N, HG), jnp.float32),
    ],
)
def _sc_scatter(msg_hbm, idx_hbm, zeros_hbm, out_hbm, idx_v, rows_v, agg_sh):
    cid = lax.axis_index("c")
    sid = lax.axis_index("s")
    wid = sid * _NC + cid
    rows_per_tile = N // _NS
    pltpu.sync_copy(zeros_hbm.at[pl.ds(sid * rows_per_tile, rows_per_tile)],
                    agg_sh.at[pl.ds(sid * rows_per_tile, rows_per_tile)])
    plsc.subcore_barrier()
    pltpu.sync_copy(msg_hbm.at[pl.ds(wid * _EPW, _EPW)], rows_v)
    pltpu.sync_copy(idx_hbm.at[pl.ds(wid * _CH, _CH)], idx_v)
    for j in range(_CH):
        pltpu.sync_copy(rows_v.at[pl.ds(j * 128, 128)],
                        agg_sh.at[idx_v.at[j]], add=True)
    plsc.subcore_barrier()
    pltpu.sync_copy(agg_sh.at[pl.ds(sid * rows_per_tile, rows_per_tile)],
                    out_hbm.at[pl.ds(cid * N + sid * rows_per_tile,
                                     rows_per_tile)])


def _msg_body(ehaug_ref, hsrc_ref, w_ref, out_ref, g_ref):
    hs = hsrc_ref[:, :HP]
    for k in range(NK):
        g_ref[:, k * HP:(k + 1) * HP] = ehaug_ref[:, k:k + 1] * hs
    msg = jnp.dot(g_ref[...], w_ref[...],
                  preferred_element_type=jnp.float32,
                  precision=jax.lax.Precision.HIGHEST)
    out_ref[...] = jnp.concatenate(
        [msg, jnp.zeros((msg.shape[0], HG - HP), jnp.float32)], axis=1)


def _msg_kernel(ehaug, hsrc, wstack):
    return pl.pallas_call(
        _msg_body,
        grid=(E // BE,),
        in_specs=[
            pl.BlockSpec((BE, 16), lambda i: (i, 0)),
            pl.BlockSpec((BE, HG), lambda i: (i, 0)),
            pl.BlockSpec((NK * HP, HP), lambda i: (0, 0)),
        ],
        out_specs=pl.BlockSpec((BE, HG), lambda i: (i, 0)),
        out_shape=jax.ShapeDtypeStruct((E, HG), jnp.float32),
        scratch_shapes=[pltpu.VMEM((BE, NK * HP), jnp.float32)],
    )(ehaug, hsrc, wstack)


def _gru_body(h_ref, agg_ref, root_ref, wih_ref, whh_ref, bih_ref, bhh_ref,
              out_ref):
    h = h_ref[:, :HP]
    agg = agg_ref[:N, :HP] + agg_ref[N:, :HP]
    m = jax.nn.relu(jnp.dot(h, root_ref[...],
                            preferred_element_type=jnp.float32) + agg)
    gi = jnp.dot(m, wih_ref[...], preferred_element_type=jnp.float32) \
        + bih_ref[...]
    gh = jnp.dot(h, whh_ref[...], preferred_element_type=jnp.float32) \
        + bhh_ref[...]
    ir, iz, inn = gi[:, :HP], gi[:, HP:2 * HP], gi[:, 2 * HP:]
    hr, hz, hn = gh[:, :HP], gh[:, HP:2 * HP], gh[:, 2 * HP:]
    r = jax.nn.sigmoid(ir + hr)
    z = jax.nn.sigmoid(iz + hz)
    n = jnp.tanh(inn + r * hn)
    hnew = (1.0 - z) * n + z * h
    out_ref[...] = jnp.concatenate(
        [hnew, jnp.zeros((hnew.shape[0], HG - HP), jnp.float32)], axis=1)


def _gru_kernel(h, agg, root, wih, whh, bih, bhh):
    return pl.pallas_call(
        _gru_body,
        out_shape=jax.ShapeDtypeStruct((N, HG), jnp.float32),
    )(h, agg, root, wih, whh, bih, bhh)


def _prologue_body(x_ref, w_ref, b_ref, out_ref):
    h0 = jax.nn.relu(
        jnp.dot(x_ref[...], w_ref[...], preferred_element_type=jnp.float32)
        + b_ref[...])
    out_ref[...] = jnp.concatenate(
        [h0, jnp.zeros((h0.shape[0], HG - HP), jnp.float32)], axis=1)


def _lin_relu(x, w, b, m, nout):
    return pl.pallas_call(
        _prologue_body,
        out_shape=jax.ShapeDtypeStruct((m, HG), jnp.float32),
    )(x, w, b)


def _ehaug_body(ea_ref, w_ref, b_ref, out_ref):
    eh = jax.nn.relu(
        jnp.dot(ea_ref[...], w_ref[...], preferred_element_type=jnp.float32)
        + b_ref[...])
    col = jax.lax.broadcasted_iota(jnp.int32, eh.shape, 1)
    eh_r = eh.astype(jnp.bfloat16).astype(jnp.float32)
    out_ref[...] = eh_r + jnp.where(col == 10, 1.0, 0.0)


def _ehaug_kernel(ea_pad, w, b):
    return pl.pallas_call(
        _ehaug_body,
        grid=(E // BE,),
        in_specs=[
            pl.BlockSpec((BE, 16), lambda i: (i, 0)),
            pl.BlockSpec((16, 16), lambda i: (0, 0)),
            pl.BlockSpec((1, 16), lambda i: (0, 0)),
        ],
        out_specs=pl.BlockSpec((BE, 16), lambda i: (i, 0)),
        out_shape=jax.ShapeDtypeStruct((E, 16), jnp.float32),
    )(ea_pad, w, b)


def _pad2(w, r, c):
    return jnp.zeros((r, c), w.dtype).at[:w.shape[0], :w.shape[1]].set(w)


def _pack_mpnn(p):
    q = {}
    q['lin0_w'] = _pad2(p['lin0_w'], HP, HP)
    q['lin0_b'] = _pad2(p['lin0_b'][None, :], 1, HP)
    q['e1_w'] = _pad2(p['e1_w'], 16, 16)
    q['e1_b'] = _pad2(p['e1_b'][None, :], 1, 16)
    e2w_r = p['e2_w'].astype(jnp.bfloat16).astype(jnp.float32)
    wk = jnp.concatenate([e2w_r, p['e2_b'][None, :]], axis=0)
    wk = wk.reshape(NK, H, H)
    q['wstack'] = jnp.zeros((NK, HP, HP), jnp.float32) \
        .at[:, :H, :H].set(wk).reshape(NK * HP, HP)
    q['root_w'] = _pad2(p['root_w'], HP, HP)
    wih = p['gru_wih'].T.reshape(H, 3, H)
    q['wih'] = jnp.zeros((HP, 3, HP), jnp.float32) \
        .at[:H, :, :H].set(wih).reshape(HP, 3 * HP)
    whh = p['gru_whh'].T.reshape(H, 3, H)
    q['whh'] = jnp.zeros((HP, 3, HP), jnp.float32) \
        .at[:H, :, :H].set(whh).reshape(HP, 3 * HP)
    bih = p['gru_bih'].reshape(3, H)
    q['bih'] = jnp.zeros((1, 3, HP), jnp.float32) \
        .at[0, :, :H].set(bih).reshape(1, 3 * HP)
    bhh = p['gru_bhh'].reshape(3, H)
    q['bhh'] = jnp.zeros((1, 3, HP), jnp.float32) \
        .at[0, :, :H].set(bhh).reshape(1, 3 * HP)
    return q


def _mpnn(x_pad, ea_pad, src2d, dst2d, zeros_nhp, q):
    h = _lin_relu(x_pad, q['lin0_w'], q['lin0_b'], N, HP)
    ehaug = _ehaug_kernel(ea_pad, q['e1_w'], q['e1_b'])
    for _ in range(NSTEP):
        hsrc = _sc_gather(h, src2d)
        msg = _msg_kernel(ehaug, hsrc, q['wstack'])
        agg2 = _sc_scatter(msg, dst2d, zeros_nhp)
        h = _gru_kernel(h, agg2, q['root_w'], q['wih'], q['whh'],
                        q['bih'], q['bhh'])
    return h[:, :H]


def _normalize(x):
    return x / jnp.maximum(jnp.linalg.norm(x, axis=1, keepdims=True), 1e-12)


def _seg_softmax(e, seg, nb):
    m = jax.ops.segment_max(e, seg, num_segments=nb)
    m = jnp.where(jnp.isfinite(m), m, 0.0)
    ex = jnp.exp(e - m[seg])
    s = jax.ops.segment_sum(ex, seg, num_segments=nb)
    return ex / (s[seg] + 1e-16)


def _set2set(x, seg, nb, p):
    d = x.shape[1]
    q_star = jnp.zeros((nb, 2 * d), dtype=x.dtype)
    h = jnp.zeros((nb, d), dtype=x.dtype)
    cst = jnp.zeros((nb, d), dtype=x.dtype)
    for _ in range(S2S_STEPS):
        gates = q_star @ p['wih'].T + p['bih'] + h @ p['whh'].T + p['bhh']
        i, f, g, o = jnp.split(gates, 4, axis=1)
        i = jax.nn.sigmoid(i)
        f = jax.nn.sigmoid(f)
        g = jnp.tanh(g)
        o = jax.nn.sigmoid(o)
        cst = f * cst + i * g
        h = o * jnp.tanh(cst)
        e = jnp.sum(x * h[seg], axis=1)
        a = _seg_softmax(e, seg, nb)
        r = jax.ops.segment_sum(a[:, None] * x, seg, num_segments=nb)
        q_star = jnp.concatenate([h, r], axis=1)
    return q_star


def _seg_mean_std(x, seg, nb):
    ones = jnp.ones((x.shape[0], 1), dtype=x.dtype)
    cnt = jax.ops.segment_sum(ones, seg, num_segments=nb)
    s = jax.ops.segment_sum(x, seg, num_segments=nb)
    mean = s / jnp.maximum(cnt, 1.0)
    s2 = jax.ops.segment_sum(x * x, seg, num_segments=nb)
    var = (s2 - cnt * mean * mean) / jnp.maximum(cnt - 1.0, 1.0)
    std = jnp.sqrt(jnp.maximum(var, 0.0))
    return mean, std


def _contrastive(a, b, tau):
    na = jnp.linalg.norm(a, axis=1)
    nb_ = jnp.linalg.norm(b, axis=1)
    sim = (a @ b.T) / (na[:, None] * nb_[None, :])
    sim = jnp.exp(sim / tau)
    pos = jnp.diag(sim)
    loss = pos / (sim.sum(axis=1) - pos)
    return -jnp.log(loss).mean()


def _pred_mlp_body(x_ref, w1_ref, b1_ref, w2_ref, b2_ref, w3_ref, b3_ref,
                   out_ref):
    h1 = jax.nn.relu(x_ref[...] @ w1_ref[...] + b1_ref[...])
    h2 = jax.nn.relu(h1 @ w2_ref[...] + b2_ref[...])
    out_ref[...] = h2 @ w3_ref[...] + b3_ref[...]


def _pred_mlp(final, pr):
    return pl.pallas_call(
        _pred_mlp_body,
        out_shape=jax.ShapeDtypeStruct((final.shape[0], 1), final.dtype),
    )(final, pr['p1_w'], pr['p1_b'][None, :], pr['p2_w'], pr['p2_b'][None, :],
      pr['p3_w'], pr['p3_b'][None, :])


def kernel(solute_x, solute_edge_index, solute_edge_attr, solute_batch,
           solvent_x, solvent_edge_index, solvent_edge_attr, solvent_batch,
           params):
    sb, vb = solute_batch, solvent_batch
    rng = jax.random.key(7)

    def padx(x):
        return jnp.zeros((N, HP), x.dtype).at[:, :H].set(x)

    def padea(ea):
        return jnp.zeros((E, 16), ea.dtype).at[:, :10].set(ea)

    qs = _pack_mpnn(params['solute'])
    qv = _pack_mpnn(params['solvent'])
    zeros_nhp = jnp.zeros((N, HG), jnp.float32)
    hs = _mpnn(padx(solute_x), padea(solute_edge_attr),
               solute_edge_index[0].reshape(E // 128, 128),
               solute_edge_index[1].reshape(E // 128, 128), zeros_nhp, qs)
    hv = _mpnn(padx(solvent_x), padea(solvent_edge_attr),
               solvent_edge_index[0].reshape(E // 128, 128),
               solvent_edge_index[1].reshape(E // 128, 128), zeros_nhp, qv)
    fs = _normalize(hs)
    fv = _normalize(hv)
    len_map = (sb[:, None] == vb[None, :]).astype(fs.dtype)
    imap = (fs @ fv.T) * len_map
    v_prime = imap.T @ fs
    s_prime = imap @ fv
    fs = jnp.concatenate([fs, s_prime], axis=1)
    fv = jnp.concatenate([fv, v_prime], axis=1)
    c = params['compressor']
    a = fs @ c['c1_w'] + c['c1_b']
    mu = a.mean(axis=0)
    var = a.var(axis=0)
    a = (a - mu) / jnp.sqrt(var + 1e-5) * c['bn_g'] + c['bn_b']
    a = jax.nn.relu(a)
    p_logit = a @ c['c2_w'] + c['c2_b']
    k1, k2 = jax.random.split(rng)
    bias = 1e-4
    u = jax.random.uniform(k1, p_logit.shape, dtype=p_logit.dtype)
    eps = (2.0 * bias - 1.0) * u + (1.0 - bias)
    gate = jax.nn.sigmoid(jnp.log(eps) - jnp.log(1.0 - eps) + p_logit)
    lam_pos = gate.reshape(-1, 1)
    lam_neg = 1.0 - lam_pos
    mean_g, std_g = _seg_mean_std(fs, sb, NB)
    mean_n = mean_g[sb]
    std_n = std_g[sb]
    noisy_mean = lam_pos * fs + lam_neg * mean_n
    noisy_std = lam_neg * std_n
    noise = jax.random.uniform(k2, noisy_mean.shape, dtype=noisy_mean.dtype)
    noisy = noisy_mean + noise * noisy_std
    sub_s = _set2set(noisy, sb, NB, params['s2s_solute'])
    eps2 = 1e-07
    kl1 = jax.ops.segment_sum(((noisy_std ** 2) / ((std_n + eps2) ** 2)).mean(axis=1), sb, num_segments=NB).reshape(-1, 1)
    kl2 = jax.ops.segment_sum(((noisy_mean - mean_n) / (std_n + eps2)) ** 2, sb, num_segments=NB)
    KL_Loss = (0.5 * kl1 + kl2).mean()
    sub_v = _set2set(fv, vb, NB, params['s2s_solvent'])
    cont = _contrastive(sub_s, sub_v, 1.0)
    final = jnp.concatenate([sub_s, sub_v], axis=1)
    preds = _pred_mlp(final, params['pred'])
    return preds, KL_Loss, cont


# speedup vs baseline: 2.8725x; 1.9266x over previous
import functools

import jax
import jax.numpy as jnp
from jax import lax
from jax.experimental import pallas as pl
from jax.experimental.pallas import tpu as pltpu
from jax.experimental.pallas import tpu_sc as plsc

H = 56
HP = 64
D2 = 112
NSTEP = 3
S2S_STEPS = 2
NB = 64
N = 2048
E = 16384
NK = 11
HG = 128
BE = 2048


_NC = 2
_NS = 16
_NW = _NC * _NS
_EPW = E // _NW
_CH = _EPW // 128

@functools.lru_cache(maxsize=None)
def _sc_gather_fn():
    mesh = plsc.VectorSubcoreMesh(core_axis_name="c", subcore_axis_name="s")
    return functools.partial(
        pl.kernel,
        mesh=mesh,
        out_type=jax.ShapeDtypeStruct((E, HG), jnp.float32),
        scratch_types=[
            pltpu.VMEM((_CH, 128), jnp.int32),
            pltpu.VMEM((_EPW, HG), jnp.float32),
            pltpu.SemaphoreType.DMA,
        ],
    )(_sc_gather_body)


def _sc_gather(h, idx2d):
    return _sc_gather_fn()(h, idx2d)


def _sc_gather_body(h_hbm, idx_hbm, out_hbm, idx_v, rows_v, sem):
    wid = lax.axis_index("s") * _NC + lax.axis_index("c")
    pltpu.sync_copy(idx_hbm.at[pl.ds(wid * _CH, _CH)], idx_v)
    copies = []
    for j in range(_CH):
        copies.append(pltpu.async_copy(
            h_hbm.at[idx_v.at[j]], rows_v.at[pl.ds(j * 128, 128)], sem))
    for cp in copies:
        cp.wait()
    pltpu.sync_copy(rows_v, out_hbm.at[pl.ds(wid * _EPW, _EPW)])


@functools.lru_cache(maxsize=None)
def _sc_scatter_fn():
    mesh = plsc.VectorSubcoreMesh(core_axis_name="c", subcore_axis_name="s")
    return functools.partial(
        pl.kernel,
        mesh=mesh,
        out_type=jax.ShapeDtypeStruct((_NC * N, HG), jnp.float32),
        scratch_types=[
            pltpu.VMEM((_CH, 128), jnp.int32),
            pltpu.VMEM((_EPW, HG), jnp.float32),
            pltpu.VMEM_SHARED((N, HG), jnp.float32),
        ],
    )(_sc_scatter_body)


def _sc_scatter(msg, idx2d, zeros):
    return _sc_scatter_fn()(msg, idx2d, zeros)


def _sc_scatter_body(msg_hbm, idx_hbm, zeros_hbm, out_hbm, idx_v, rows_v,
                     agg_sh):
    cid = lax.axis_index("c")
    sid = lax.axis_index("s")
    wid = sid * _NC + cid
    rows_per_tile = N // _NS
    pltpu.sync_copy(zeros_hbm.at[pl.ds(sid * rows_per_tile, rows_per_tile)],
                    agg_sh.at[pl.ds(sid * rows_per_tile, rows_per_tile)])
    plsc.subcore_barrier()
    pltpu.sync_copy(msg_hbm.at[pl.ds(wid * _EPW, _EPW)], rows_v)
    pltpu.sync_copy(idx_hbm.at[pl.ds(wid * _CH, _CH)], idx_v)
    for j in range(_CH):
        pltpu.sync_copy(rows_v.at[pl.ds(j * 128, 128)],
                        agg_sh.at[idx_v.at[j]], add=True)
    plsc.subcore_barrier()
    pltpu.sync_copy(agg_sh.at[pl.ds(sid * rows_per_tile, rows_per_tile)],
                    out_hbm.at[pl.ds(cid * N + sid * rows_per_tile,
                                     rows_per_tile)])


def _msg_body(ehaug_ref, hsrc_ref, w_ref, out_ref, g_ref):
    hs = hsrc_ref[:, :HP]
    for k in range(NK):
        g_ref[:, k * HP:(k + 1) * HP] = ehaug_ref[:, k:k + 1] * hs
    msg = jnp.dot(g_ref[...], w_ref[...],
                  preferred_element_type=jnp.float32,
                  precision=jax.lax.Precision.HIGHEST)
    out_ref[...] = jnp.concatenate(
        [msg, jnp.zeros((msg.shape[0], HG - HP), jnp.float32)], axis=1)


def _msg_kernel(ehaug, hsrc, wstack):
    return pl.pallas_call(
        _msg_body,
        grid=(E // BE,),
        in_specs=[
            pl.BlockSpec((BE, 16), lambda i: (i, 0)),
            pl.BlockSpec((BE, HG), lambda i: (i, 0)),
            pl.BlockSpec((NK * HP, HP), lambda i: (0, 0)),
        ],
        out_specs=pl.BlockSpec((BE, HG), lambda i: (i, 0)),
        out_shape=jax.ShapeDtypeStruct((E, HG), jnp.float32),
        scratch_shapes=[pltpu.VMEM((BE, NK * HP), jnp.float32)],
    )(ehaug, hsrc, wstack)


def _gru_body(h_ref, agg_ref, root_ref, wih_ref, whh_ref, bih_ref, bhh_ref,
              out_ref):
    h = h_ref[:, :HP]
    agg = agg_ref[:N, :HP] + agg_ref[N:, :HP]
    m = jax.nn.relu(jnp.dot(h, root_ref[...],
                            preferred_element_type=jnp.float32) + agg)
    gi = jnp.dot(m, wih_ref[...], preferred_element_type=jnp.float32) \
        + bih_ref[...]
    gh = jnp.dot(h, whh_ref[...], preferred_element_type=jnp.float32) \
        + bhh_ref[...]
    ir, iz, inn = gi[:, :HP], gi[:, HP:2 * HP], gi[:, 2 * HP:]
    hr, hz, hn = gh[:, :HP], gh[:, HP:2 * HP], gh[:, 2 * HP:]
    r = jax.nn.sigmoid(ir + hr)
    z = jax.nn.sigmoid(iz + hz)
    n = jnp.tanh(inn + r * hn)
    hnew = (1.0 - z) * n + z * h
    out_ref[...] = jnp.concatenate(
        [hnew, jnp.zeros((hnew.shape[0], HG - HP), jnp.float32)], axis=1)


def _gru_kernel(h, agg, root, wih, whh, bih, bhh):
    return pl.pallas_call(
        _gru_body,
        out_shape=jax.ShapeDtypeStruct((N, HG), jnp.float32),
    )(h, agg, root, wih, whh, bih, bhh)


def _prologue_body(x_ref, w_ref, b_ref, out_ref):
    h0 = jax.nn.relu(
        jnp.dot(x_ref[...], w_ref[...], preferred_element_type=jnp.float32)
        + b_ref[...])
    out_ref[...] = jnp.concatenate(
        [h0, jnp.zeros((h0.shape[0], HG - HP), jnp.float32)], axis=1)


def _lin_relu(x, w, b, m, nout):
    return pl.pallas_call(
        _prologue_body,
        out_shape=jax.ShapeDtypeStruct((m, HG), jnp.float32),
    )(x, w, b)


def _ehaug_body(ea_ref, w_ref, b_ref, out_ref):
    eh = jax.nn.relu(
        jnp.dot(ea_ref[...], w_ref[...], preferred_element_type=jnp.float32)
        + b_ref[...])
    col = jax.lax.broadcasted_iota(jnp.int32, eh.shape, 1)
    eh_r = eh.astype(jnp.bfloat16).astype(jnp.float32)
    out_ref[...] = eh_r + jnp.where(col == 10, 1.0, 0.0)


def _ehaug_kernel(ea_pad, w, b):
    return pl.pallas_call(
        _ehaug_body,
        grid=(E // BE,),
        in_specs=[
            pl.BlockSpec((BE, 16), lambda i: (i, 0)),
            pl.BlockSpec((16, 16), lambda i: (0, 0)),
            pl.BlockSpec((1, 16), lambda i: (0, 0)),
        ],
        out_specs=pl.BlockSpec((BE, 16), lambda i: (i, 0)),
        out_shape=jax.ShapeDtypeStruct((E, 16), jnp.float32),
    )(ea_pad, w, b)


NEGINF = -3.0e38


def _seg_softmax_nodemajor(e, mn):
    em = jnp.where(mn > 0.0, e, NEGINF)
    mx = jnp.max(em, axis=0, keepdims=True)
    mx = jnp.where(mx > NEGINF / 2.0, mx, 0.0)
    mseg = jnp.sum(mn * mx, axis=1, keepdims=True)
    ex = jnp.exp(e - mseg)
    sg = jnp.sum(mn * ex, axis=0, keepdims=True)
    sseg = jnp.sum(mn * sg, axis=1, keepdims=True)
    return ex / (sseg + 1e-16)


def _set2set_blk(x, mn, wih, whh, bih, bhh):
    q_star = jnp.zeros((NB, 4 * HP), jnp.float32)
    h = jnp.zeros((NB, 2 * HP), jnp.float32)
    cst = jnp.zeros((NB, 2 * HP), jnp.float32)
    hi = jax.lax.Precision.HIGHEST
    for _ in range(S2S_STEPS):
        gates = (jnp.dot(q_star, wih, preferred_element_type=jnp.float32)
                 + bih
                 + jnp.dot(h, whh, preferred_element_type=jnp.float32)
                 + bhh)
        i = jax.nn.sigmoid(gates[:, :2 * HP])
        f = jax.nn.sigmoid(gates[:, 2 * HP:4 * HP])
        g = jnp.tanh(gates[:, 4 * HP:6 * HP])
        o = jax.nn.sigmoid(gates[:, 6 * HP:])
        cst = f * cst + i * g
        h = o * jnp.tanh(cst)
        hseg = jnp.dot(mn, h, preferred_element_type=jnp.float32,
                       precision=hi)
        e = jnp.sum(x * hseg, axis=1, keepdims=True)
        a = _seg_softmax_nodemajor(e, mn)
        r = jax.lax.dot_general(mn, a * x, (((0,), (0,)), ((), ())),
                                preferred_element_type=jnp.float32,
                                precision=hi)
        q_star = jnp.concatenate([h, r], axis=1)
    return q_star


def _post_body(hs_ref, hv_ref, sbc_ref, vbc_ref, lb_ref, noise_ref,
               c1_ref, c1b_ref, bng_ref, bnb_ref, c2_ref, c2b_ref,
               ssw_ref, ssu_ref, ssbi_ref, ssbh_ref,
               svw_ref, svu_ref, svbi_ref, svbh_ref,
               p1_ref, b1_ref, p2_ref, b2_ref, p3_ref, b3_ref,
               preds_ref, kl_ref, cont_ref):
    hi = jax.lax.Precision.HIGHEST
    hs = hs_ref[:, :HP]
    hv = hv_ref[:, :HP]
    sbc = sbc_ref[...]
    vbc = vbc_ref[...]
    fs = hs / jnp.maximum(
        jnp.sqrt(jnp.sum(hs * hs, axis=1, keepdims=True)), 1e-12)
    fv = hv / jnp.maximum(
        jnp.sqrt(jnp.sum(hv * hv, axis=1, keepdims=True)), 1e-12)
    eq = sbc == jnp.transpose(vbc)
    imap = jax.lax.dot_general(fs, fv, (((1,), (1,)), ((), ())),
                               preferred_element_type=jnp.float32)
    imap = jnp.where(eq, imap, 0.0)
    s_prime = jnp.dot(imap, fv, preferred_element_type=jnp.float32)
    v_prime = jax.lax.dot_general(imap, fs, (((0,), (0,)), ((), ())),
                                  preferred_element_type=jnp.float32)
    fs2 = jnp.concatenate([fs, s_prime], axis=1)
    fv2 = jnp.concatenate([fv, v_prime], axis=1)
    a = jnp.dot(fs2, c1_ref[...], preferred_element_type=jnp.float32) \
        + c1b_ref[...]
    mu = jnp.mean(a, axis=0, keepdims=True)
    var = jnp.mean((a - mu) * (a - mu), axis=0, keepdims=True)
    an = (a - mu) / jnp.sqrt(var + 1e-5) * bng_ref[...] + bnb_ref[...]
    an = jax.nn.relu(an)
    p_logit = jnp.dot(an, c2_ref[...], preferred_element_type=jnp.float32) \
        + c2b_ref[...]
    gate = jax.nn.sigmoid(lb_ref[...] + p_logit)
    lam_neg = 1.0 - gate
    gid = jax.lax.broadcasted_iota(jnp.int32, (N, NB), 1)
    mn_s = (sbc == gid).astype(jnp.float32)
    mn_v = (vbc == gid).astype(jnp.float32)
    cnt = jnp.sum(mn_s, axis=0, keepdims=True)
    ssum = jax.lax.dot_general(mn_s, fs2, (((0,), (0,)), ((), ())),
                               preferred_element_type=jnp.float32,
                               precision=hi)
    s2sum = jax.lax.dot_general(mn_s, fs2 * fs2, (((0,), (0,)), ((), ())),
                                preferred_element_type=jnp.float32,
                                precision=hi)
    cnt_c = jnp.transpose(cnt)
    mean_g = ssum / jnp.maximum(cnt_c, 1.0)
    var_g = (s2sum - cnt_c * mean_g * mean_g) / jnp.maximum(cnt_c - 1.0, 1.0)
    std_g = jnp.sqrt(jnp.maximum(var_g, 0.0))
    mean_n = jnp.dot(mn_s, mean_g, preferred_element_type=jnp.float32,
                     precision=hi)
    std_n = jnp.dot(mn_s, std_g, preferred_element_type=jnp.float32,
                    precision=hi)
    noisy_mean = gate * fs2 + lam_neg * mean_n
    noisy_std = lam_neg * std_n
    noisy = noisy_mean + noise_ref[...] * noisy_std
    eps2 = 1e-07
    rat = (noisy_std * noisy_std) / ((std_n + eps2) * (std_n + eps2))
    kl1n = jnp.sum(rat, axis=1, keepdims=True) / float(D2)
    kl1 = jax.lax.dot_general(mn_s, kl1n, (((0,), (0,)), ((), ())),
                              preferred_element_type=jnp.float32,
                              precision=hi)
    dev = (noisy_mean - mean_n) / (std_n + eps2)
    kl2 = jax.lax.dot_general(mn_s, dev * dev, (((0,), (0,)), ((), ())),
                              preferred_element_type=jnp.float32,
                              precision=hi)
    col = jax.lax.broadcasted_iota(jnp.int32, (1, 2 * HP), 1)
    realc = jnp.logical_or(col < H, jnp.logical_and(col >= HP, col < HP + H))
    realf = realc.astype(jnp.float32)
    kl_ref[...] = (jnp.sum((0.5 * kl1 + kl2) * realf)
                   / float(NB * D2)).reshape(1, 1)
    sub_s = _set2set_blk(noisy, mn_s, ssw_ref[...], ssu_ref[...],
                         ssbi_ref[...], ssbh_ref[...])
    sub_v = _set2set_blk(fv2, mn_v, svw_ref[...], svu_ref[...],
                         svbi_ref[...], svbh_ref[...])
    na = jnp.sqrt(jnp.sum(sub_s * sub_s, axis=1, keepdims=True))
    nb_ = jnp.sqrt(jnp.sum(sub_v * sub_v, axis=1, keepdims=True))
    sim = jax.lax.dot_general(sub_s, sub_v, (((1,), (1,)), ((), ())),
                              preferred_element_type=jnp.float32)
    sim = jnp.exp(sim / (na * jnp.transpose(nb_)))
    r_i = jax.lax.broadcasted_iota(jnp.int32, (NB, NB), 0)
    c_i = jax.lax.broadcasted_iota(jnp.int32, (NB, NB), 1)
    diag = (r_i == c_i).astype(jnp.float32)
    pos = jnp.sum(sim * diag, axis=1, keepdims=True)
    tot = jnp.sum(sim, axis=1, keepdims=True)
    cont_ref[...] = (-jnp.mean(jnp.log(pos / (tot - pos)))).reshape(1, 1)
    final = jnp.concatenate([sub_s, sub_v], axis=1)
    h1 = jax.nn.relu(jnp.dot(final, p1_ref[...],
                             preferred_element_type=jnp.float32) + b1_ref[...])
    h2 = jax.nn.relu(jnp.dot(h1, p2_ref[...],
                             preferred_element_type=jnp.float32) + b2_ref[...])
    preds_ref[...] = jnp.dot(h2, p3_ref[...],
                             preferred_element_type=jnp.float32) + b3_ref[...]


def _pad2(w, r, c):
    return jnp.zeros((r, c), w.dtype).at[:w.shape[0], :w.shape[1]].set(w)


def _pack_mpnn(p):
    q = {}
    q['lin0_w'] = _pad2(p['lin0_w'], HP, HP)
    q['lin0_b'] = _pad2(p['lin0_b'][None, :], 1, HP)
    q['e1_w'] = _pad2(p['e1_w'], 16, 16)
    q['e1_b'] = _pad2(p['e1_b'][None, :], 1, 16)
    e2w_r = p['e2_w'].astype(jnp.bfloat16).astype(jnp.float32)
    wk = jnp.concatenate([e2w_r, p['e2_b'][None, :]], axis=0)
    wk = wk.reshape(NK, H, H)
    q['wstack'] = jnp.zeros((NK, HP, HP), jnp.float32) \
        .at[:, :H, :H].set(wk).reshape(NK * HP, HP)
    q['root_w'] = _pad2(p['root_w'], HP, HP)
    wih = p['gru_wih'].T.reshape(H, 3, H)
    q['wih'] = jnp.zeros((HP, 3, HP), jnp.float32) \
        .at[:H, :, :H].set(wih).reshape(HP, 3 * HP)
    whh = p['gru_whh'].T.reshape(H, 3, H)
    q['whh'] = jnp.zeros((HP, 3, HP), jnp.float32) \
        .at[:H, :, :H].set(whh).reshape(HP, 3 * HP)
    bih = p['gru_bih'].reshape(3, H)
    q['bih'] = jnp.zeros((1, 3, HP), jnp.float32) \
        .at[0, :, :H].set(bih).reshape(1, 3 * HP)
    bhh = p['gru_bhh'].reshape(3, H)
    q['bhh'] = jnp.zeros((1, 3, HP), jnp.float32) \
        .at[0, :, :H].set(bhh).reshape(1, 3 * HP)
    return q


def _expand56(m, nbi, nbo):
    m = m.reshape(nbi, H, nbo, H)
    out = jnp.zeros((nbi, HP, nbo, HP), jnp.float32)
    out = out.at[:, :H, :, :H].set(m)
    return out.reshape(nbi * HP, nbo * HP)


def _expand56_rows(m, nbi):
    c = m.shape[1]
    m = m.reshape(nbi, H, c)
    out = jnp.zeros((nbi, HP, c), jnp.float32)
    out = out.at[:, :H, :].set(m)
    return out.reshape(nbi * HP, c)


def _expand56_vec(v, nb_):
    v = v.reshape(nb_, H)
    out = jnp.zeros((nb_, HP), jnp.float32)
    out = out.at[:, :H].set(v)
    return out.reshape(1, nb_ * HP)


def _pack_post(params):
    c = params['compressor']
    pr = params['pred']
    s2s_s = params['s2s_solute']
    s2s_v = params['s2s_solvent']
    pk = {}
    pk['c1'] = _expand56(c['c1_w'], 2, 1)
    pk['c1b'] = _expand56_vec(c['c1_b'], 1)
    pk['bng'] = _expand56_vec(c['bn_g'], 1)
    pk['bnb'] = _expand56_vec(c['bn_b'], 1)
    pk['c2'] = _expand56_rows(c['c2_w'], 1)
    pk['c2b'] = c['c2_b'].reshape(1, 1)
    for tag, s in (('ss', s2s_s), ('sv', s2s_v)):
        pk[tag + 'w'] = _expand56(s['wih'].T, 4, 8)
        pk[tag + 'u'] = _expand56(s['whh'].T, 2, 8)
        pk[tag + 'bi'] = _expand56_vec(s['bih'], 8)
        pk[tag + 'bh'] = _expand56_vec(s['bhh'], 8)
    pk['p1'] = _expand56_rows(pr['p1_w'], 8)
    pk['b1'] = pr['p1_b'][None, :]
    pk['p2'] = pr['p2_w']
    pk['b2'] = pr['p2_b'][None, :]
    pk['p3'] = pr['p3_w']
    pk['b3'] = pr['p3_b'][None, :]
    return pk


def _post_kernel(hs, hv, sbc, vbc, lb, noise_p, pk):
    return pl.pallas_call(
        _post_body,
        out_shape=(
            jax.ShapeDtypeStruct((NB, 1), jnp.float32),
            jax.ShapeDtypeStruct((1, 1), jnp.float32),
            jax.ShapeDtypeStruct((1, 1), jnp.float32),
        ),
    )(hs, hv, sbc, vbc, lb, noise_p,
      pk['c1'], pk['c1b'], pk['bng'], pk['bnb'], pk['c2'], pk['c2b'],
      pk['ssw'], pk['ssu'], pk['ssbi'], pk['ssbh'],
      pk['svw'], pk['svu'], pk['svbi'], pk['svbh'],
      pk['p1'], pk['b1'], pk['p2'], pk['b2'], pk['p3'], pk['b3'])


def _mpnn(x_pad, ea_pad, src2d, dst2d, zeros_nhp, q):
    h = _lin_relu(x_pad, q['lin0_w'], q['lin0_b'], N, HP)
    ehaug = _ehaug_kernel(ea_pad, q['e1_w'], q['e1_b'])
    for _ in range(NSTEP):
        hsrc = _sc_gather(h, src2d)
        msg = _msg_kernel(ehaug, hsrc, q['wstack'])
        agg2 = _sc_scatter(msg, dst2d, zeros_nhp)
        h = _gru_kernel(h, agg2, q['root_w'], q['wih'], q['whh'],
                        q['bih'], q['bhh'])
    return h


def _normalize(x):
    return x / jnp.maximum(jnp.linalg.norm(x, axis=1, keepdims=True), 1e-12)


def _seg_softmax(e, seg, nb):
    m = jax.ops.segment_max(e, seg, num_segments=nb)
    m = jnp.where(jnp.isfinite(m), m, 0.0)
    ex = jnp.exp(e - m[seg])
    s = jax.ops.segment_sum(ex, seg, num_segments=nb)
    return ex / (s[seg] + 1e-16)


def _set2set(x, seg, nb, p):
    d = x.shape[1]
    q_star = jnp.zeros((nb, 2 * d), dtype=x.dtype)
    h = jnp.zeros((nb, d), dtype=x.dtype)
    cst = jnp.zeros((nb, d), dtype=x.dtype)
    for _ in range(S2S_STEPS):
        gates = q_star @ p['wih'].T + p['bih'] + h @ p['whh'].T + p['bhh']
        i, f, g, o = jnp.split(gates, 4, axis=1)
        i = jax.nn.sigmoid(i)
        f = jax.nn.sigmoid(f)
        g = jnp.tanh(g)
        o = jax.nn.sigmoid(o)
        cst = f * cst + i * g
        h = o * jnp.tanh(cst)
        e = jnp.sum(x * h[seg], axis=1)
        a = _seg_softmax(e, seg, nb)
        r = jax.ops.segment_sum(a[:, None] * x, seg, num_segments=nb)
        q_star = jnp.concatenate([h, r], axis=1)
    return q_star


def _seg_mean_std(x, seg, nb):
    ones = jnp.ones((x.shape[0], 1), dtype=x.dtype)
    cnt = jax.ops.segment_sum(ones, seg, num_segments=nb)
    s = jax.ops.segment_sum(x, seg, num_segments=nb)
    mean = s / jnp.maximum(cnt, 1.0)
    s2 = jax.ops.segment_sum(x * x, seg, num_segments=nb)
    var = (s2 - cnt * mean * mean) / jnp.maximum(cnt - 1.0, 1.0)
    std = jnp.sqrt(jnp.maximum(var, 0.0))
    return mean, std


def _contrastive(a, b, tau):
    na = jnp.linalg.norm(a, axis=1)
    nb_ = jnp.linalg.norm(b, axis=1)
    sim = (a @ b.T) / (na[:, None] * nb_[None, :])
    sim = jnp.exp(sim / tau)
    pos = jnp.diag(sim)
    loss = pos / (sim.sum(axis=1) - pos)
    return -jnp.log(loss).mean()


def _pred_mlp_body(x_ref, w1_ref, b1_ref, w2_ref, b2_ref, w3_ref, b3_ref,
                   out_ref):
    h1 = jax.nn.relu(x_ref[...] @ w1_ref[...] + b1_ref[...])
    h2 = jax.nn.relu(h1 @ w2_ref[...] + b2_ref[...])
    out_ref[...] = h2 @ w3_ref[...] + b3_ref[...]


def _pred_mlp(final, pr):
    return pl.pallas_call(
        _pred_mlp_body,
        out_shape=jax.ShapeDtypeStruct((final.shape[0], 1), final.dtype),
    )(final, pr['p1_w'], pr['p1_b'][None, :], pr['p2_w'], pr['p2_b'][None, :],
      pr['p3_w'], pr['p3_b'][None, :])


def kernel(solute_x, solute_edge_index, solute_edge_attr, solute_batch,
           solvent_x, solvent_edge_index, solvent_edge_attr, solvent_batch,
           params):
    sb, vb = solute_batch, solvent_batch
    rng = jax.random.key(7)

    def padx(x):
        return jnp.zeros((N, HP), x.dtype).at[:, :H].set(x)

    def padea(ea):
        return jnp.zeros((E, 16), ea.dtype).at[:, :10].set(ea)

    qs = _pack_mpnn(params['solute'])
    qv = _pack_mpnn(params['solvent'])
    pk = _pack_post(params)
    zeros_nhp = jnp.zeros((N, HG), jnp.float32)
    hs = _mpnn(padx(solute_x), padea(solute_edge_attr),
               solute_edge_index[0].reshape(E // 128, 128),
               solute_edge_index[1].reshape(E // 128, 128), zeros_nhp, qs)
    hv = _mpnn(padx(solvent_x), padea(solvent_edge_attr),
               solvent_edge_index[0].reshape(E // 128, 128),
               solvent_edge_index[1].reshape(E // 128, 128), zeros_nhp, qv)
    k1, k2 = jax.random.split(rng)
    bias = 1e-4
    u = jax.random.uniform(k1, (N, 1), dtype=jnp.float32)
    eps = (2.0 * bias - 1.0) * u + (1.0 - bias)
    lb = jnp.log(eps) - jnp.log(1.0 - eps)
    noise = jax.random.uniform(k2, (N, D2), dtype=jnp.float32)
    noise_p = jnp.zeros((N, 2 * HP), jnp.float32)
    noise_p = noise_p.at[:, :H].set(noise[:, :H])
    noise_p = noise_p.at[:, HP:HP + H].set(noise[:, H:])
    preds, kl, cont = _post_kernel(hs, hv,
                                   sb.reshape(N, 1), vb.reshape(N, 1),
                                   lb, noise_p, pk)
    return preds, kl[0, 0], cont[0, 0]
